# baseline (speedup 1.0000x reference)
import functools

import jax
import jax.numpy as jnp
from jax import lax
from jax.experimental import pallas as pl
from jax.experimental.pallas import tpu as pltpu
from jax.experimental.pallas import tpu_sc as plsc

_NC = 2
_NS = 16
_NW = _NC * _NS
_C = 128
_R = 2


@functools.lru_cache(maxsize=None)
def _make_gather(S, B, V, D):
    n_blocks_total = S * (B // _C)
    n_per_w = n_blocks_total // _NW
    tb_n = B // _C
    mesh = plsc.VectorSubcoreMesh(core_axis_name="c", subcore_axis_name="s")

    @functools.partial(
        pl.kernel,
        out_type=jax.ShapeDtypeStruct((S, D // 8, tb_n, 8, _C), jnp.float32),
        mesh=mesh,
        scratch_types=[
            pltpu.VMEM((n_per_w // 8, 8, _C), jnp.int32),
            pltpu.VMEM((_R, _C), jnp.int32),
            pltpu.VMEM((_R, _C, 2 * D), jnp.float32),
            pltpu.VMEM((_R, D, _C), jnp.float32),
            pltpu.SemaphoreType.DMA((_R,)),
            pltpu.SemaphoreType.DMA((_R,)),
        ],
        compiler_params=pltpu.CompilerParams(needs_layout_passes=False),
    )
    def gather_kernel(xt_hbm, t2_hbm, out_hbm, idx_v, gl_v, rows_v, tbuf_v,
                      gsem, osem):
        wid = lax.axis_index("s") * _NC + lax.axis_index("c")
        pltpu.sync_copy(xt_hbm.at[wid], idx_v)
        viota = lax.iota(jnp.int32, 16)

        def fire_gather(t, b):
            tq, tr = t // 8, t % 8
            for k in range(_C // 16):
                gl_v[b, pl.ds(k * 16, 16)] = (
                    idx_v[tq, tr, pl.ds(k * 16, 16)] >> 1
                )
            pltpu.async_copy(t2_hbm.at[gl_v.at[b]], rows_v.at[b], gsem.at[b])

        def wait_gather(b):
            pltpu.make_async_copy(
                t2_hbm.at[gl_v.at[b]], rows_v.at[b], gsem.at[b]
            ).wait()

        def wait_owrites(b):
            for td in range(D // 8):
                pltpu.make_async_copy(
                    tbuf_v.at[b, pl.ds(td * 8, 8)],
                    out_hbm.at[0, td, 0],
                    osem.at[b],
                ).wait()

        def transpose_block(t, b):
            tq, tr = t // 8, t % 8
            rows2d = rows_v.at[b]
            rowvs = [viota + k * 16 for k in range(_C // 16)]
            parvs = [
                (idx_v[tq, tr, pl.ds(k * 16, 16)] & 1) * D
                for k in range(_C // 16)
            ]

            @plsc.parallel_loop(0, D, unroll=8)
            def _(d):
                for k in range(_C // 16):
                    vec = plsc.load_gather(rows2d, [rowvs[k], parvs[k] + d])
                    tbuf_v[b, d, pl.ds(k * 16, 16)] = vec

        for b in range(_R):
            fire_gather(b, b)

        @pl.loop(0, n_per_w)
        def _(t):
            b = lax.rem(t, _R)
            j = wid * n_per_w + t
            s = j // tb_n
            tb = j % tb_n

            wait_gather(b)

            @pl.when(t >= _R)
            def _():
                wait_owrites(b)

            transpose_block(t, b)

            for td in range(D // 8):
                pltpu.async_copy(
                    tbuf_v.at[b, pl.ds(td * 8, 8)],
                    out_hbm.at[s, td, tb],
                    osem.at[b],
                )

            @pl.when(t + _R < n_per_w)
            def _():
                fire_gather(t + _R, b)

        for b in range(_R):
            wait_owrites(b)

    return gather_kernel


def kernel(x, table):
    B, S = x.shape
    V, D = table.shape
    t2 = table.reshape(V // 2, 2 * D)
    xt = jnp.transpose(x).reshape(_NW, (B * S) // (_NW * _C * 8), 8, _C)
    out5 = _make_gather(S, B, V, D)(xt, t2)
    out = (
        out5.transpose(0, 1, 3, 2, 4)
        .reshape(S, D, B)
        .transpose(2, 0, 1)
    )
    return out


# speedup vs baseline: 1.0269x; 1.0269x over previous
import functools

import jax
import jax.numpy as jnp
from jax import lax
from jax.experimental import pallas as pl
from jax.experimental.pallas import tpu as pltpu
from jax.experimental.pallas import tpu_sc as plsc

_NC = 2
_NS = 16
_NW = _NC * _NS
_C = 128
_R = 4


@functools.lru_cache(maxsize=None)
def _make_gather(N, V, D):
    n_per_w = N // _NW
    n_chunks = n_per_w // _C
    mesh = plsc.VectorSubcoreMesh(core_axis_name="c", subcore_axis_name="s")

    @functools.partial(
        pl.kernel,
        out_type=jax.ShapeDtypeStruct((N, D), jnp.float32),
        mesh=mesh,
        scratch_types=[
            pltpu.VMEM((n_chunks, _C), jnp.int32),
            pltpu.VMEM((_R, _C, D), jnp.float32),
        ]
        + [pltpu.SemaphoreType.DMA] * (2 * _R),
        compiler_params=pltpu.CompilerParams(use_tc_tiling_on_sc=False),
    )
    def gather_kernel(idx_hbm, table_hbm, out_hbm, idx_v, rows_v, *sems):
        gsem, osem = sems[:_R], sems[_R:]
        wid = lax.axis_index("s") * _NC + lax.axis_index("c")
        base = wid * n_per_w
        pltpu.sync_copy(idx_hbm.at[wid], idx_v)

        for b in range(_R):
            pltpu.async_copy(table_hbm.at[idx_v.at[b]], rows_v.at[b], gsem[b])

        @pl.loop(0, n_chunks, step=_R)
        def _(i0):
            for b in range(_R):
                i = i0 + b
                pltpu.make_async_copy(
                    table_hbm.at[idx_v.at[i]], rows_v.at[b], gsem[b]
                ).wait()
                wdesc = pltpu.async_copy(
                    rows_v.at[b], out_hbm.at[pl.ds(base + i * _C, _C)], osem[b]
                )

                @pl.when(i + _R < n_chunks)
                def _():
                    wdesc.wait()
                    pltpu.async_copy(
                        table_hbm.at[idx_v.at[i + _R]], rows_v.at[b], gsem[b]
                    )

        for b in range(_R):
            pltpu.make_async_copy(
                rows_v.at[b], out_hbm.at[pl.ds(base, _C)], osem[b]
            ).wait()

    return gather_kernel


def kernel(x, table):
    B, S = x.shape
    V, D = table.shape
    N = B * S
    idx = x.reshape(_NW, (N // _NW) // _C, _C).astype(jnp.int32)
    out = _make_gather(N, V, D)(idx, table)
    return out.reshape(B, S, D)
